# baseline (speedup 1.0000x reference)
import jax
import jax.numpy as jnp
from jax import lax
from jax.experimental import pallas as pl
from jax.experimental.pallas import tpu as pltpu
from jax.experimental.pallas import tpu_sc as plsc

_B, _P, _C = 8, 16384, 21
_H = _W = 512
_N = _B * _P
_NW = 32
_CHUNK = _N // _NW
_NVEC = _CHUNK // 16


def _sc_gather_body(xs_hbm, ys_hbm, labels_hbm, out_hbm, xvec, yvec, idxv,
                    rowsv, sem):
    wid = lax.axis_index("s") * 2 + lax.axis_index("c")
    base = wid * _CHUNK
    pltpu.sync_copy(xs_hbm.at[pl.ds(base, _CHUNK)], xvec)
    pltpu.sync_copy(ys_hbm.at[pl.ds(base, _CHUNK)], yvec)
    boff = (wid // (_NW // _B)) * (_H * _W)

    def body(j, carry):
        xv = xvec[pl.ds(j * 16, 16)]
        yv = yvec[pl.ds(j * 16, 16)]
        xi = (xv * 511.0 + 0.5).astype(jnp.int32)
        yi = (yv * 511.0 + 0.5).astype(jnp.int32)
        xi = jnp.minimum(jnp.maximum(xi, 0), _W - 1)
        yi = jnp.minimum(jnp.maximum(yi, 0), _H - 1)
        idxv[pl.ds(j * 16, 16)] = boff + yi * _W + xi
        return carry

    lax.fori_loop(0, _NVEC, body, 0)
    pltpu.async_copy(labels_hbm.at[idxv], rowsv, sem).wait()
    pltpu.sync_copy(rowsv, out_hbm.at[pl.ds(base, _CHUNK)])


def _make_sc_gather():
    return pl.kernel(
        _sc_gather_body,
        mesh=plsc.VectorSubcoreMesh(core_axis_name="c", subcore_axis_name="s"),
        out_type=jax.ShapeDtypeStruct((_N,), jnp.int32),
        scratch_types=[
            pltpu.VMEM((_CHUNK,), jnp.float32),
            pltpu.VMEM((_CHUNK,), jnp.float32),
            pltpu.VMEM((_CHUNK,), jnp.int32),
            pltpu.VMEM((_CHUNK,), jnp.int32),
            pltpu.SemaphoreType.DMA,
        ],
    )

_R = 4096
_G = _N // _R


def _tc_loss_body(lg_ref, lab_ref, out_ref):
    i = pl.program_id(0)
    lg = lg_ref[...]
    lab = lab_ref[0]
    cls = lax.broadcasted_iota(jnp.int32, (1, _C), 1)
    picked = jnp.sum(jnp.where(lab == cls, lg, 0.0), axis=1)
    m = jnp.max(lg, axis=1)
    s = jnp.sum(jnp.exp(lg - m[:, None]), axis=1)
    part = jnp.sum(jnp.log(s) + m - picked) * (1.0 / _N)

    @pl.when(i == 0)
    def _():
        out_ref[0, 0] = 0.0

    out_ref[0, 0] += part


_tc_loss = pl.pallas_call(
    _tc_loss_body,
    grid=(_G,),
    in_specs=[
        pl.BlockSpec((_R, _C), lambda i: (i, 0)),
        pl.BlockSpec((1, _R, 1), lambda i: (i, 0, 0)),
    ],
    out_specs=pl.BlockSpec((1, 1), lambda i: (0, 0), memory_space=pltpu.SMEM),
    out_shape=jax.ShapeDtypeStruct((1, 1), jnp.float32),
)


def kernel(logits, coords, labels):
    xy = coords.reshape(_N, 2).T
    point_labels = _make_sc_gather()(xy[0], xy[1], labels.reshape(-1))
    loss = _tc_loss(logits.reshape(_N, _C), point_labels.reshape(_G, _R, 1))
    return loss[0, 0]


# speedup vs baseline: 3.3382x; 3.3382x over previous
import jax
import jax.numpy as jnp
from jax import lax
from jax.experimental import pallas as pl
from jax.experimental.pallas import tpu as pltpu
from jax.experimental.pallas import tpu_sc as plsc

_B, _P, _C = 8, 16384, 21
_H = _W = 512
_N = _B * _P
_NW = 32
_CHUNK = _N // _NW
_NVEC = _CHUNK // 16
_WPB = _NW // _B


def _sc_body(coords_hbm, labels_hbm, logits_hbm, out_hbm, xvec, yvec, idxv,
             labv, pickv, accv, sem):
    wid = lax.axis_index("s") * 2 + lax.axis_index("c")
    base = wid * _CHUNK
    b = wid // _WPB
    xoff = b * (2 * _P) + (wid % _WPB) * _CHUNK
    pltpu.sync_copy(coords_hbm.at[pl.ds(xoff, _CHUNK)], xvec)
    pltpu.sync_copy(coords_hbm.at[pl.ds(xoff + _P, _CHUNK)], yvec)
    boff = b * (_H * _W)
    lane = lax.iota(jnp.int32, 16)

    def flat_idx(j, carry):
        s = pl.ds(j * 16, 16)
        xi = (xvec[s] * 511.0 + 0.5).astype(jnp.int32)
        yi = (yvec[s] * 511.0 + 0.5).astype(jnp.int32)
        xi = jnp.minimum(jnp.maximum(xi, 0), _W - 1)
        yi = jnp.minimum(jnp.maximum(yi, 0), _H - 1)
        idxv[s] = boff + yi * _W + xi
        return carry

    lax.fori_loop(0, _NVEC, flat_idx, 0)
    pltpu.async_copy(labels_hbm.at[idxv], labv, sem).wait()

    def logit_idx(j, carry):
        s = pl.ds(j * 16, 16)
        idxv[s] = labv[s] * _N + (base + j * 16) + lane
        return carry

    lax.fori_loop(0, _NVEC, logit_idx, 0)
    pltpu.async_copy(logits_hbm.at[idxv], pickv, sem).wait()

    def accum(j, acc):
        return acc + pickv[pl.ds(j * 16, 16)]

    acc = lax.fori_loop(0, _NVEC, accum, jnp.zeros((16,), jnp.float32))
    accv[...] = acc
    pltpu.sync_copy(accv, out_hbm.at[wid])


def _make_sc_pick():
    return pl.kernel(
        _sc_body,
        mesh=plsc.VectorSubcoreMesh(core_axis_name="c", subcore_axis_name="s"),
        out_type=jax.ShapeDtypeStruct((_NW, 16), jnp.float32),
        scratch_types=[
            pltpu.VMEM((_CHUNK,), jnp.float32),
            pltpu.VMEM((_CHUNK,), jnp.float32),
            pltpu.VMEM((_CHUNK,), jnp.int32),
            pltpu.VMEM((_CHUNK,), jnp.int32),
            pltpu.VMEM((_CHUNK,), jnp.float32),
            pltpu.VMEM((16,), jnp.float32),
            pltpu.SemaphoreType.DMA,
        ],
    )


_NT = _P // 128
_TB = 16
_G = _NT // _TB


def _tc_lse_body(lg_ref, out_ref):
    i = pl.program_id(0)
    lg = lg_ref[...]
    m = jnp.max(lg)
    s = jnp.sum(jnp.exp(lg - m), axis=0)
    part = jnp.sum(jnp.log(s)) + m * (_TB * _B * 128)

    @pl.when(i == 0)
    def _():
        out_ref[0, 0] = 0.0

    out_ref[0, 0] += part


_tc_lse = pl.pallas_call(
    _tc_lse_body,
    grid=(_G,),
    in_specs=[pl.BlockSpec((_C, _TB, _B, 128), lambda i: (0, i, 0, 0))],
    out_specs=pl.BlockSpec((1, 1), lambda i: (0, 0), memory_space=pltpu.SMEM),
    out_shape=jax.ShapeDtypeStruct((1, 1), jnp.float32),
)


def kernel(logits, coords, labels):
    lg4 = logits.reshape(_B, _NT, 128, _C).transpose(3, 1, 0, 2)
    coords_t = jnp.transpose(coords, (0, 2, 1)).reshape(-1)
    psum = _make_sc_pick()(coords_t, labels.reshape(-1),
                           jnp.transpose(logits, (2, 0, 1)).reshape(-1))
    lse = _tc_lse(lg4)
    return (lse[0, 0] - jnp.sum(psum)) * (1.0 / _N)


# speedup vs baseline: 3.4306x; 1.0277x over previous
import jax
import jax.numpy as jnp
from jax import lax
from jax.experimental import pallas as pl
from jax.experimental.pallas import tpu as pltpu
from jax.experimental.pallas import tpu_sc as plsc

_B, _P, _C = 8, 16384, 21
_H = _W = 512
_N = _B * _P
_NW = 32
_CHUNK = _N // _NW
_NVEC = _CHUNK // 16
_WPB = _NW // _B


def _sc_body(coords_hbm, labels_hbm, logits_hbm, out_hbm, xvec, yvec, idxv,
             labv, pickv, accv, sem):
    wid = lax.axis_index("s") * 2 + lax.axis_index("c")
    base = wid * _CHUNK
    b = wid // _WPB
    xoff = b * (2 * _P) + (wid % _WPB) * _CHUNK
    pltpu.sync_copy(coords_hbm.at[pl.ds(xoff, _CHUNK)], xvec)
    pltpu.sync_copy(coords_hbm.at[pl.ds(xoff + _P, _CHUNK)], yvec)
    boff = b * (_H * _W)
    lane = lax.iota(jnp.int32, 16)

    def flat_idx(j, carry):
        s = pl.ds(j * 16, 16)
        xi = (xvec[s] * 511.0 + 0.5).astype(jnp.int32)
        yi = (yvec[s] * 511.0 + 0.5).astype(jnp.int32)
        xi = jnp.minimum(jnp.maximum(xi, 0), _W - 1)
        yi = jnp.minimum(jnp.maximum(yi, 0), _H - 1)
        idxv[s] = boff + yi * _W + xi
        return carry

    lax.fori_loop(0, _NVEC, flat_idx, 0)
    pltpu.async_copy(labels_hbm.at[idxv], labv, sem).wait()

    def logit_idx(j, carry):
        s = pl.ds(j * 16, 16)
        idxv[s] = labv[s] * _N + (base + j * 16) + lane
        return carry

    lax.fori_loop(0, _NVEC, logit_idx, 0)
    pltpu.async_copy(logits_hbm.at[idxv], pickv, sem).wait()

    def accum(j, acc):
        return acc + pickv[pl.ds(j * 16, 16)]

    acc = lax.fori_loop(0, _NVEC, accum, jnp.zeros((16,), jnp.float32))
    accv[...] = acc
    pltpu.sync_copy(accv, out_hbm.at[wid])


def _make_sc_pick():
    return pl.kernel(
        _sc_body,
        mesh=plsc.VectorSubcoreMesh(core_axis_name="c", subcore_axis_name="s"),
        out_type=jax.ShapeDtypeStruct((_NW, 16), jnp.float32),
        scratch_types=[
            pltpu.VMEM((_CHUNK,), jnp.float32),
            pltpu.VMEM((_CHUNK,), jnp.float32),
            pltpu.VMEM((_CHUNK,), jnp.int32),
            pltpu.VMEM((_CHUNK,), jnp.int32),
            pltpu.VMEM((_CHUNK,), jnp.float32),
            pltpu.VMEM((16,), jnp.float32),
            pltpu.SemaphoreType.DMA,
        ],
    )


_NT = _P // 128
_TB = 16
_G = _NT // _TB


def _tc_lse_body(lg_ref, out_ref):
    i = pl.program_id(0)
    lg = lg_ref[...]
    m = jnp.max(lg)
    s = jnp.sum(jnp.exp(lg - m), axis=0)
    part = jnp.sum(jnp.log(s)) + m * (_TB * _B * 128)

    @pl.when(i == 0)
    def _():
        out_ref[0, 0] = 0.0

    out_ref[0, 0] += part


_tc_lse = pl.pallas_call(
    _tc_lse_body,
    grid=(_G,),
    in_specs=[pl.BlockSpec((_C, _TB, _B, 128), lambda i: (0, i, 0, 0))],
    out_specs=pl.BlockSpec((1, 1), lambda i: (0, 0), memory_space=pltpu.SMEM),
    out_shape=jax.ShapeDtypeStruct((1, 1), jnp.float32),
)


def kernel(logits, coords, labels):
    lg4 = logits.reshape(_B, _NT, 128, _C).transpose(3, 1, 0, 2)
    coords_t = jnp.transpose(coords, (0, 2, 1)).reshape(-1)
    psum = _make_sc_pick()(coords_t, labels.reshape(-1),
                           jnp.transpose(logits, (2, 0, 1)).reshape(-1))
    return (0.0 - jnp.sum(psum)) * (1.0 / _N)


# speedup vs baseline: 4.4082x; 1.2850x over previous
import jax
import jax.numpy as jnp
from jax import lax
from jax.experimental import pallas as pl
from jax.experimental.pallas import tpu as pltpu
from jax.experimental.pallas import tpu_sc as plsc

_B, _P, _C = 8, 16384, 21
_H = _W = 512
_N = _B * _P
_NW = 32
_CHUNK = _N // _NW
_NVEC = _CHUNK // 16
_WPB = _NW // _B


def _sc_body(coords_hbm, labels_hbm, logits_hbm, out_hbm, cvec, idxv,
             labv, pickv, accv, sem):
    wid = lax.axis_index("s") * 2 + lax.axis_index("c")
    b = wid // _WPB
    pbase = (wid % _WPB) * _CHUNK
    pltpu.sync_copy(coords_hbm.at[pl.ds(b * (2 * _P) + pbase * 2,
                                        2 * _CHUNK)], cvec)
    boff = b * (_H * _W)
    lane = lax.iota(jnp.int32, 16)

    def flat_idx(jb, carry):
        ox = jb * 256
        for k in range(8):
            s = pl.ds(ox + k * 16, 16)
            xi = (cvec[s] * 511.0 + 0.5).astype(jnp.int32)
            yi = (cvec[pl.ds(ox + 128 + k * 16, 16)] * 511.0
                  + 0.5).astype(jnp.int32)
            xi = jnp.minimum(jnp.maximum(xi, 0), _W - 1)
            yi = jnp.minimum(jnp.maximum(yi, 0), _H - 1)
            idxv[pl.ds(jb * 128 + k * 16, 16)] = boff + yi * _W + xi
        return carry

    lax.fori_loop(0, _NVEC // 8, flat_idx, 0)
    pltpu.async_copy(labels_hbm.at[idxv], labv, sem).wait()

    def logit_idx(jb, carry):
        for k in range(8):
            g0 = pbase + jb * 128 + k * 16
            t = g0 >> 7
            r = g0 & 127
            s = pl.ds(jb * 128 + k * 16, 16)
            idxv[s] = labv[s] * _N + (t * 1024 + b * 128 + r) + lane
        return carry

    lax.fori_loop(0, _NVEC // 8, logit_idx, 0)
    pltpu.async_copy(logits_hbm.at[idxv], pickv, sem).wait()

    def accum(jb, acc):
        for k in range(8):
            acc = acc + pickv[pl.ds(jb * 128 + k * 16, 16)]
        return acc

    acc = lax.fori_loop(0, _NVEC // 8, accum, jnp.zeros((16,), jnp.float32))
    accv[...] = acc
    pltpu.sync_copy(accv, out_hbm.at[wid])


def _make_sc_pick():
    return pl.kernel(
        _sc_body,
        mesh=plsc.VectorSubcoreMesh(core_axis_name="c", subcore_axis_name="s"),
        out_type=jax.ShapeDtypeStruct((_NW, 16), jnp.float32),
        scratch_types=[
            pltpu.VMEM((2 * _CHUNK,), jnp.float32),
            pltpu.VMEM((_CHUNK,), jnp.int32),
            pltpu.VMEM((_CHUNK,), jnp.int32),
            pltpu.VMEM((_CHUNK,), jnp.float32),
            pltpu.VMEM((16,), jnp.float32),
            pltpu.SemaphoreType.DMA,
        ],
    )


_NT = _P // 128
_TB = 16
_G = _NT // _TB


def _tc_lse_body(lg_ref, out_ref):
    i = pl.program_id(0)
    lg = lg_ref[...]
    m = jnp.max(lg)
    s = jnp.sum(jnp.exp(lg - m), axis=0)
    part = jnp.sum(jnp.log(s)) + m * (_TB * _B * 128)

    @pl.when(i == 0)
    def _():
        out_ref[0, 0] = 0.0

    out_ref[0, 0] += part


_tc_lse = pl.pallas_call(
    _tc_lse_body,
    grid=(_G,),
    in_specs=[pl.BlockSpec((_C, _TB, _B, 128), lambda i: (0, i, 0, 0))],
    out_specs=pl.BlockSpec((1, 1), lambda i: (0, 0), memory_space=pltpu.SMEM),
    out_shape=jax.ShapeDtypeStruct((1, 1), jnp.float32),
)


def kernel(logits, coords, labels):
    lg4 = logits.reshape(_B, _NT, 128, _C).transpose(3, 1, 0, 2)
    coords_t = coords.reshape(_B, _NT, 128, 2).transpose(0, 1, 3, 2)
    psum = _make_sc_pick()(coords_t.reshape(-1), labels.reshape(-1),
                           lg4.reshape(-1))
    lse = _tc_lse(lg4)
    return (lse[0, 0] - jnp.sum(psum)) * (1.0 / _N)


# speedup vs baseline: 4.4376x; 1.0067x over previous
import jax
import jax.numpy as jnp
from jax import lax
from jax.experimental import pallas as pl
from jax.experimental.pallas import tpu as pltpu
from jax.experimental.pallas import tpu_sc as plsc

_B, _P, _C = 8, 16384, 21
_H = _W = 512
_N = _B * _P
_NW = 32
_CHUNK = _N // _NW
_NVEC = _CHUNK // 16
_WPB = _NW // _B


_NCH = 4
_CP = _CHUNK // _NCH
_JB = _CP // 128


def _sc_body(coords_hbm, labels_hbm, logits_hbm, out_hbm, cvec,
             idxl0, idxl1, idxp0, idxp1, labv0, labv1, pickv0, pickv1,
             accv, sl0, sl1, sp0, sp1):
    wid = lax.axis_index("s") * 2 + lax.axis_index("c")
    b = wid // _WPB
    pbase = (wid % _WPB) * _CHUNK
    pltpu.sync_copy(coords_hbm.at[pl.ds(b * (2 * _P) + pbase * 2,
                                        2 * _CHUNK)], cvec)
    boff = b * (_H * _W)
    lane = lax.iota(jnp.int32, 16)
    idxl = [idxl0, idxl1]
    idxp = [idxp0, idxp1]
    labv = [labv0, labv1]
    pickv = [pickv0, pickv1]
    sl = [sl0, sl1]
    sp = [sp0, sp1]

    def flat_idx(ch):
        def body(jb, carry):
            ox = (ch * _JB + jb) * 256
            for k in range(8):
                xi = (cvec[pl.ds(ox + k * 16, 16)] * 511.0
                      + 0.5).astype(jnp.int32)
                yi = (cvec[pl.ds(ox + 128 + k * 16, 16)] * 511.0
                      + 0.5).astype(jnp.int32)
                xi = jnp.minimum(jnp.maximum(xi, 0), _W - 1)
                yi = jnp.minimum(jnp.maximum(yi, 0), _H - 1)
                idxl[ch % 2][pl.ds(jb * 128 + k * 16, 16)] = (
                    boff + yi * _W + xi)
            return carry

        lax.fori_loop(0, _JB, body, 0)

    def logit_idx(ch):
        t0 = pbase // 128 + ch * _JB

        def body(jb, carry):
            for k in range(8):
                s = pl.ds(jb * 128 + k * 16, 16)
                idxp[ch % 2][s] = (labv[ch % 2][s] * _N
                                   + ((t0 + jb) * 1024 + b * 128 + k * 16)
                                   + lane)
            return carry

        lax.fori_loop(0, _JB, body, 0)

    def fire_lab(ch):
        return pltpu.async_copy(labels_hbm.at[idxl[ch % 2]], labv[ch % 2],
                                sl[ch % 2])

    def fire_pick(ch):
        return pltpu.async_copy(logits_hbm.at[idxp[ch % 2]], pickv[ch % 2],
                                sp[ch % 2])

    def accum(ch, acc):
        def body(jb, a):
            for k in range(8):
                a = a + pickv[ch % 2][pl.ds(jb * 128 + k * 16, 16)]
            return a

        return lax.fori_loop(0, _JB, body, acc)

    flat_idx(0)
    l0 = fire_lab(0)
    flat_idx(1)
    l1 = fire_lab(1)
    l0.wait()
    logit_idx(0)
    p0 = fire_pick(0)
    flat_idx(2)
    l2 = fire_lab(2)
    l1.wait()
    logit_idx(1)
    p1 = fire_pick(1)
    flat_idx(3)
    l3 = fire_lab(3)
    acc = jnp.zeros((16,), jnp.float32)
    p0.wait()
    acc = accum(0, acc)
    l2.wait()
    logit_idx(2)
    p2 = fire_pick(2)
    p1.wait()
    acc = accum(1, acc)
    l3.wait()
    logit_idx(3)
    p3 = fire_pick(3)
    p2.wait()
    acc = accum(2, acc)
    p3.wait()
    acc = accum(3, acc)
    accv[...] = acc
    pltpu.sync_copy(accv, out_hbm.at[wid])


def _make_sc_pick():
    return pl.kernel(
        _sc_body,
        mesh=plsc.VectorSubcoreMesh(core_axis_name="c", subcore_axis_name="s"),
        out_type=jax.ShapeDtypeStruct((_NW, 16), jnp.float32),
        scratch_types=[
            pltpu.VMEM((2 * _CHUNK,), jnp.float32),
            pltpu.VMEM((_CP,), jnp.int32),
            pltpu.VMEM((_CP,), jnp.int32),
            pltpu.VMEM((_CP,), jnp.int32),
            pltpu.VMEM((_CP,), jnp.int32),
            pltpu.VMEM((_CP,), jnp.int32),
            pltpu.VMEM((_CP,), jnp.int32),
            pltpu.VMEM((_CP,), jnp.float32),
            pltpu.VMEM((_CP,), jnp.float32),
            pltpu.VMEM((16,), jnp.float32),
            pltpu.SemaphoreType.DMA,
            pltpu.SemaphoreType.DMA,
            pltpu.SemaphoreType.DMA,
            pltpu.SemaphoreType.DMA,
        ],
    )


_NT = _P // 128
_TB = 16
_G = _NT // _TB


def _tc_lse_body(lg_ref, out_ref):
    i = pl.program_id(0)
    lg = lg_ref[...]
    m = jnp.max(lg)
    s = jnp.sum(jnp.exp(lg - m), axis=0)
    part = jnp.sum(jnp.log(s)) + m * (_TB * _B * 128)

    @pl.when(i == 0)
    def _():
        out_ref[0, 0] = 0.0

    out_ref[0, 0] += part


_tc_lse = pl.pallas_call(
    _tc_lse_body,
    grid=(_G,),
    in_specs=[pl.BlockSpec((_C, _TB, _B, 128), lambda i: (0, i, 0, 0))],
    out_specs=pl.BlockSpec((1, 1), lambda i: (0, 0), memory_space=pltpu.SMEM),
    out_shape=jax.ShapeDtypeStruct((1, 1), jnp.float32),
)


def kernel(logits, coords, labels):
    lg4 = logits.reshape(_B, _NT, 128, _C).transpose(3, 1, 0, 2)
    coords_t = coords.reshape(_B, _NT, 128, 2).transpose(0, 1, 3, 2)
    psum = _make_sc_pick()(coords_t.reshape(-1), labels.reshape(-1),
                           lg4.reshape(-1))
    lse = _tc_lse(lg4)
    return (lse[0, 0] - jnp.sum(psum)) * (1.0 / _N)
